# baseline (speedup 1.0000x reference)
import functools

import jax
import jax.numpy as jnp
from jax.experimental import pallas as pl
from jax.experimental.pallas import tpu as pltpu

_VOCAB = 8192
_DIM = 32
_COMMIT = 0.25
_BM = 1024
_BK = 2048
_NKB = _VOCAB // _BK


def _vq_body(x_ref, w_ref, x2_ref, w2_ref, idx_ref, qst_ref, mse_ref):
    x = x_ref[0]
    x2 = x2_ref[...]

    def dist_step(kb, carry):
        best, bidx = carry
        w_blk = w_ref[pl.ds(kb * _BK, _BK), :]
        w2_blk = w2_ref[:, pl.ds(kb * _BK, _BK)]
        xw = jax.lax.dot_general(
            x, w_blk, (((1,), (1,)), ((), ())),
            preferred_element_type=jnp.float32)
        d2 = (x2 + w2_blk) - 2.0 * xw
        d = jnp.sqrt(jnp.maximum(d2, 0.0))
        lmin = jnp.min(d, axis=1, keepdims=True)
        kiota = jax.lax.broadcasted_iota(jnp.int32, (_BM, _BK), 1)
        lidx = jnp.min(jnp.where(d == lmin, kiota, _VOCAB),
                       axis=1, keepdims=True) + kb * _BK
        upd = lmin < best
        return (jnp.where(upd, lmin, best), jnp.where(upd, lidx, bidx))

    best0 = jnp.full((_BM, 1), jnp.inf, dtype=jnp.float32)
    bidx0 = jnp.zeros((_BM, 1), dtype=jnp.int32)
    best, bidx = jax.lax.fori_loop(0, _NKB, dist_step, (best0, bidx0))

    idx_ref[...] = bidx
    mse_ref[0] = jnp.sum(best * best, axis=0, keepdims=True)

    def gather_step(kb, q):
        w_blk = w_ref[pl.ds(kb * _BK, _BK), :]
        kiota = jax.lax.broadcasted_iota(jnp.int32, (_BM, _BK), 1)
        oh = (kiota == (bidx - kb * _BK)).astype(jnp.float32)
        return q + jax.lax.dot_general(
            oh, w_blk, (((1,), (0,)), ((), ())),
            precision=jax.lax.Precision.HIGHEST,
            preferred_element_type=jnp.float32)

    q = jax.lax.fori_loop(0, _NKB, gather_step,
                          jnp.zeros((_BM, _DIM), dtype=jnp.float32))
    qst_ref[0] = x + (q - x)


@jax.jit
def kernel(x, W):
    B, N, D = x.shape
    M = B * N
    x2 = jnp.sum(x * x, axis=-1, keepdims=True)
    w2 = jnp.sum(W * W, axis=-1)[None, :]
    x2f = x2.reshape(M, 1)

    grid = (M // _BM,)
    idx_flat, qst, mse_part = pl.pallas_call(
        _vq_body,
        grid=grid,
        in_specs=[
            pl.BlockSpec((1, _BM, D), lambda i: (i, 0, 0)),
            pl.BlockSpec((_VOCAB, D), lambda i: (0, 0)),
            pl.BlockSpec((_BM, 1), lambda i: (i, 0)),
            pl.BlockSpec((1, _VOCAB), lambda i: (0, 0)),
        ],
        out_specs=[
            pl.BlockSpec((_BM, 1), lambda i: (i, 0)),
            pl.BlockSpec((1, _BM, D), lambda i: (i, 0, 0)),
            pl.BlockSpec((1, 1, 1), lambda i: (i, 0, 0)),
        ],
        out_shape=[
            jax.ShapeDtypeStruct((M, 1), jnp.int32),
            jax.ShapeDtypeStruct((B, N, D), jnp.float32),
            jax.ShapeDtypeStruct((grid[0], 1, 1), jnp.float32),
        ],
    )(x.reshape(B, N, D), W, x2f, w2)

    mse = jnp.sum(mse_part) / (M * D)
    loss = mse + _COMMIT * mse
    return (qst, loss, mse, idx_flat.reshape(B, N))


# speedup vs baseline: 2.3431x; 2.3431x over previous
import jax
import jax.numpy as jnp
from jax.experimental import pallas as pl
from jax.experimental.pallas import tpu as pltpu

_VOCAB = 8192
_DIM = 32
_COMMIT = 0.25
_BM = 1024
_BK = 2048
_NKB = _VOCAB // _BK


def _vq_body(x_ref, w_ref, x2_ref, w2_ref, idx_ref, qst_ref, mse_ref, d2_scr):
    x = x_ref[0]
    x2 = x2_ref[...]

    def dist_step(kb, md2):
        w_blk = w_ref[pl.ds(kb * _BK, _BK), :]
        w2_blk = w2_ref[:, pl.ds(kb * _BK, _BK)]
        xw = jax.lax.dot_general(
            x, w_blk, (((1,), (1,)), ((), ())),
            preferred_element_type=jnp.float32)
        d2 = (x2 + w2_blk) - 2.0 * xw
        d2_scr[:, pl.ds(kb * _BK, _BK)] = d2
        return jnp.minimum(md2, jnp.min(d2, axis=1, keepdims=True))

    md2 = jax.lax.fori_loop(
        0, _NKB, dist_step, jnp.full((_BM, 1), jnp.inf, dtype=jnp.float32))
    md2 = jnp.maximum(md2, 0.0)
    m = jnp.sqrt(md2)
    mse_ref[0] = jnp.sum(m * m, axis=0, keepdims=True)

    mn = jax.lax.bitcast_convert_type(
        jax.lax.bitcast_convert_type(m, jnp.int32) + 1, jnp.float32)
    p = m * mn
    c = m * 4097.0
    mh = c - (c - m)
    ml = m - mh
    cn = mn * 4097.0
    nh = cn - (cn - mn)
    nl = mn - nh
    r = ((mh * nh - p) + mh * nl + ml * nh) + ml * nl
    U = jnp.where(
        r < 0.0,
        jax.lax.bitcast_convert_type(
            jax.lax.bitcast_convert_type(p, jnp.int32) - 1, jnp.float32),
        p)

    kiota = jax.lax.broadcasted_iota(jnp.int32, (_BM, _BK), 1)

    def idx_step(kb, gidx):
        d2 = d2_scr[:, pl.ds(kb * _BK, _BK)]
        li = jnp.min(jnp.where(d2 <= U, kiota, _BK),
                     axis=1, keepdims=True)
        upd = (gidx < 0) & (li < _BK)
        return jnp.where(upd, li + kb * _BK, gidx)

    gidx = jax.lax.fori_loop(
        0, _NKB, idx_step, jnp.full((_BM, 1), -1, dtype=jnp.int32))
    idx_ref[...] = gidx

    def gather_step(kb, q):
        w_blk = w_ref[pl.ds(kb * _BK, _BK), :]
        oh = (kiota == (gidx - kb * _BK)).astype(jnp.float32)
        return q + jax.lax.dot_general(
            oh, w_blk, (((1,), (0,)), ((), ())),
            preferred_element_type=jnp.float32)

    q = jax.lax.fori_loop(0, _NKB, gather_step,
                          jnp.zeros((_BM, _DIM), dtype=jnp.float32))
    qst_ref[0] = x + (q - x)


@jax.jit
def kernel(x, W):
    B, N, D = x.shape
    M = B * N
    x2 = jnp.sum(x * x, axis=-1, keepdims=True)
    w2 = jnp.sum(W * W, axis=-1)[None, :]
    x2f = x2.reshape(M, 1)

    grid = (M // _BM,)
    idx_flat, qst, mse_part = pl.pallas_call(
        _vq_body,
        grid=grid,
        in_specs=[
            pl.BlockSpec((1, _BM, D), lambda i: (i, 0, 0)),
            pl.BlockSpec((_VOCAB, D), lambda i: (0, 0)),
            pl.BlockSpec((_BM, 1), lambda i: (i, 0)),
            pl.BlockSpec((1, _VOCAB), lambda i: (0, 0)),
        ],
        out_specs=[
            pl.BlockSpec((_BM, 1), lambda i: (i, 0)),
            pl.BlockSpec((1, _BM, D), lambda i: (i, 0, 0)),
            pl.BlockSpec((1, 1, 1), lambda i: (i, 0, 0)),
        ],
        out_shape=[
            jax.ShapeDtypeStruct((M, 1), jnp.int32),
            jax.ShapeDtypeStruct((B, N, D), jnp.float32),
            jax.ShapeDtypeStruct((grid[0], 1, 1), jnp.float32),
        ],
        scratch_shapes=[pltpu.VMEM((_BM, _VOCAB), jnp.float32)],
    )(x.reshape(B, N, D), W, x2f, w2)

    mse = jnp.sum(mse_part) / (M * D)
    loss = mse + _COMMIT * mse
    return (qst, loss, mse, idx_flat.reshape(B, N))
